# Initial kernel scaffold; baseline (speedup 1.0000x reference)
import jax
import jax.numpy as jnp
from jax.experimental import pallas as pl


def kernel(x, edge_index, batch, W1, b1, W2, b2, W3, b3, Wl, bl):
    raise NotImplementedError("write your pallas kernel here")


# speedup vs baseline: 8.7030x; 8.7030x over previous
import functools

import jax
import jax.numpy as jnp
from jax import lax
from jax.experimental import pallas as pl
from jax.experimental.pallas import tpu as pltpu
from jax.experimental.pallas import tpu_sc as plsc

N_NODES = 10000
N_EDGES = 320000
D = 128
N_GRAPHS = 128
N_CLASSES = 10

NC = 2
NS = 16
NW = NC * NS
CH = 128
N_CHUNKS = -(-N_EDGES // (CH * NW)) * NW
PAD_E = N_CHUNKS * CH
CHUNKS_PER_W = N_CHUNKS // NW
ROW_SLC = 632
ROW_SLC_LAST = N_NODES - ROW_SLC * (NS - 1)
ACC_ROWS = N_NODES + 16

ROW_BLK = 1000
N_BLK = N_NODES // ROW_BLK


def _sc_mesh():
    return plsc.VectorSubcoreMesh(
        core_axis_name="c", subcore_axis_name="s",
        num_cores=NC, num_subcores=NS)


def _row_slice_copy(sid, fn):
    @pl.when(sid < NS - 1)
    def _():
        fn(pl.multiple_of(sid * ROW_SLC, 8), ROW_SLC)

    @pl.when(sid == NS - 1)
    def _():
        fn(ROW_SLC * (NS - 1), ROW_SLC_LAST)


def _deg_body(dstp_hbm, ones_hbm, zeros_hbm, out_hbm, ones_v, didx_v, acc):
    cid = lax.axis_index("c")
    sid = lax.axis_index("s")
    wid = cid * NS + sid

    _row_slice_copy(sid, lambda r0, nr: pltpu.sync_copy(
        zeros_hbm.at[pl.ds(r0, nr), :], acc.at[pl.ds(r0, nr), :]))
    pltpu.sync_copy(ones_hbm, ones_v)
    plsc.subcore_barrier()

    def step(c, _):
        base = (wid * CHUNKS_PER_W + c) * CH
        pltpu.sync_copy(dstp_hbm.at[pl.ds(base, CH)], didx_v)
        pltpu.sync_copy(ones_v, acc.at[didx_v], add=True)
        return 0

    lax.fori_loop(0, CHUNKS_PER_W, step, 0)
    plsc.subcore_barrier()
    _row_slice_copy(sid, lambda r0, nr: pltpu.sync_copy(
        acc.at[pl.ds(r0, nr), :], out_hbm.at[cid, pl.ds(r0, nr), :]))


def _make_deg_kernel():
    return pl.kernel(
        _deg_body,
        out_type=jax.ShapeDtypeStruct((NC, N_NODES, D), jnp.float32),
        mesh=_sc_mesh(),
        scratch_types=[
            pltpu.VMEM((CH, D), jnp.float32),
            pltpu.VMEM((CH,), jnp.int32),
            pltpu.VMEM_SHARED((ACC_ROWS, D), jnp.float32),
        ],
    )


def _edge_body(srcp_hbm, dstp_hbm, y_hbm, zeros_hbm, out_hbm,
               sidx_v, didx_v, rows_v, gsem, acc):
    cid = lax.axis_index("c")
    sid = lax.axis_index("s")
    wid = cid * NS + sid

    @pl.when(cid == 0)
    def _():
        _row_slice_copy(sid, lambda r0, nr: pltpu.sync_copy(
            y_hbm.at[pl.ds(r0, nr), :], acc.at[pl.ds(r0, nr), :]))

    @pl.when(cid != 0)
    def _():
        _row_slice_copy(sid, lambda r0, nr: pltpu.sync_copy(
            zeros_hbm.at[pl.ds(r0, nr), :], acc.at[pl.ds(r0, nr), :]))

    plsc.subcore_barrier()

    def step(c, _):
        base = (wid * CHUNKS_PER_W + c) * CH
        pltpu.sync_copy(srcp_hbm.at[pl.ds(base, CH)], sidx_v)
        pltpu.sync_copy(dstp_hbm.at[pl.ds(base, CH)], didx_v)
        pltpu.async_copy(y_hbm.at[sidx_v], rows_v, gsem).wait()
        pltpu.sync_copy(rows_v, acc.at[didx_v], add=True)
        return 0

    lax.fori_loop(0, CHUNKS_PER_W, step, 0)
    plsc.subcore_barrier()
    _row_slice_copy(sid, lambda r0, nr: pltpu.sync_copy(
        acc.at[pl.ds(r0, nr), :], out_hbm.at[cid, pl.ds(r0, nr), :]))


def _make_edge_kernel():
    return pl.kernel(
        _edge_body,
        out_type=jax.ShapeDtypeStruct((NC, N_NODES, D), jnp.float32),
        mesh=_sc_mesh(),
        scratch_types=[
            pltpu.VMEM((CH,), jnp.int32),
            pltpu.VMEM((CH,), jnp.int32),
            pltpu.VMEM((CH, D), jnp.float32),
            pltpu.SemaphoreType.DMA,
            pltpu.VMEM_SHARED((ACC_ROWS, D), jnp.float32),
        ],
    )


def _dis(d0_ref, d1_ref):
    deg = d0_ref[:, 0:1] + d1_ref[:, 0:1] + 1.0
    return lax.rsqrt(deg)


def _t1_body(x_ref, w_ref, d0_ref, d1_ref, y_ref):
    h = jnp.dot(x_ref[...], w_ref[...], preferred_element_type=jnp.float32)
    y_ref[...] = h * _dis(d0_ref, d1_ref)


def _t1(x, W1, d0, d1):
    return pl.pallas_call(
        _t1_body,
        grid=(N_BLK,),
        in_specs=[
            pl.BlockSpec((ROW_BLK, D), lambda i: (i, 0)),
            pl.BlockSpec((D, D), lambda i: (0, 0)),
            pl.BlockSpec((ROW_BLK, 16), lambda i: (i, 0)),
            pl.BlockSpec((ROW_BLK, 16), lambda i: (i, 0)),
        ],
        out_specs=pl.BlockSpec((ROW_BLK, D), lambda i: (i, 0)),
        out_shape=jax.ShapeDtypeStruct((N_NODES, D), jnp.float32),
    )(x, W1, d0, d1)


def _t2_body(p0_ref, p1_ref, d0_ref, d1_ref, b_ref, w_ref, y_ref):
    dis = _dis(d0_ref, d1_ref)
    conv = (p0_ref[...] + p1_ref[...]) * dis + b_ref[...]
    act = jnp.maximum(conv, 0.0)
    y_ref[...] = jnp.dot(act, w_ref[...],
                         preferred_element_type=jnp.float32) * dis


def _t2(p0, p1, d0, d1, b2d, W):
    return pl.pallas_call(
        _t2_body,
        grid=(N_BLK,),
        in_specs=[
            pl.BlockSpec((ROW_BLK, D), lambda i: (i, 0)),
            pl.BlockSpec((ROW_BLK, D), lambda i: (i, 0)),
            pl.BlockSpec((ROW_BLK, 16), lambda i: (i, 0)),
            pl.BlockSpec((ROW_BLK, 16), lambda i: (i, 0)),
            pl.BlockSpec((1, D), lambda i: (0, 0)),
            pl.BlockSpec((D, D), lambda i: (0, 0)),
        ],
        out_specs=pl.BlockSpec((ROW_BLK, D), lambda i: (i, 0)),
        out_shape=jax.ShapeDtypeStruct((N_NODES, D), jnp.float32),
    )(p0, p1, d0, d1, b2d, W)


def _t3_body(p0_ref, p1_ref, d0_ref, d1_ref, b_ref, batch_ref,
             sums_ref, cnts_ref):
    @pl.when(pl.program_id(0) == 0)
    def _():
        sums_ref[...] = jnp.zeros_like(sums_ref)
        cnts_ref[...] = jnp.zeros_like(cnts_ref)

    dis = _dis(d0_ref, d1_ref)
    h3 = (p0_ref[...] + p1_ref[...]) * dis + b_ref[...]
    gids = batch_ref[0]
    gcol = lax.broadcasted_iota(jnp.int32, (N_GRAPHS, 1), 0)
    mask_t = (gcol == gids).astype(jnp.float32)
    sums_ref[...] += jnp.dot(mask_t, h3, preferred_element_type=jnp.float32)
    ones_m = jnp.ones((ROW_BLK, D), jnp.float32)
    cnts_ref[...] += jnp.dot(mask_t, ones_m,
                             preferred_element_type=jnp.float32)


def _t3(p0, p1, d0, d1, b2d, batch3):
    return pl.pallas_call(
        _t3_body,
        grid=(N_BLK,),
        in_specs=[
            pl.BlockSpec((ROW_BLK, D), lambda i: (i, 0)),
            pl.BlockSpec((ROW_BLK, D), lambda i: (i, 0)),
            pl.BlockSpec((ROW_BLK, 16), lambda i: (i, 0)),
            pl.BlockSpec((ROW_BLK, 16), lambda i: (i, 0)),
            pl.BlockSpec((1, D), lambda i: (0, 0)),
            pl.BlockSpec((1, 1, ROW_BLK), lambda i: (i, 0, 0)),
        ],
        out_specs=[
            pl.BlockSpec((N_GRAPHS, D), lambda i: (0, 0)),
            pl.BlockSpec((N_GRAPHS, D), lambda i: (0, 0)),
        ],
        out_shape=[
            jax.ShapeDtypeStruct((N_GRAPHS, D), jnp.float32),
            jax.ShapeDtypeStruct((N_GRAPHS, D), jnp.float32),
        ],
    )(p0, p1, d0, d1, b2d, batch3)


def _t4_body(sums_ref, cnts_ref, wl_ref, bl_ref, out_ref):
    pooled = sums_ref[...] / jnp.maximum(cnts_ref[...], 1.0)
    out_ref[...] = jnp.dot(pooled, wl_ref[...],
                           preferred_element_type=jnp.float32) + bl_ref[...]


def _t4(sums, cnts, Wl, bl2d):
    return pl.pallas_call(
        _t4_body,
        out_shape=jax.ShapeDtypeStruct((N_GRAPHS, N_CLASSES), jnp.float32),
    )(sums, cnts, Wl, bl2d)


@jax.jit
def _run(x, edge_index, batch, W1, b1, W2, b2, W3, b3, Wl, bl):
    pad = PAD_E - N_EDGES
    srcp = jnp.concatenate([edge_index[0], jnp.zeros((pad,), jnp.int32)])
    dstp = jnp.concatenate(
        [edge_index[1], jnp.full((pad,), N_NODES, jnp.int32)])
    ones_rows = jnp.ones((CH, D), jnp.float32)
    z128 = jnp.zeros((N_NODES, D), jnp.float32)
    batch3 = batch.reshape(N_BLK, 1, ROW_BLK)
    b1r, b2r, b3r = b1.reshape(1, D), b2.reshape(1, D), b3.reshape(1, D)
    blr = bl.reshape(1, N_CLASSES)

    deg_kernel = _make_deg_kernel()
    edge_kernel = _make_edge_kernel()

    degp = deg_kernel(dstp, ones_rows, z128)
    d0, d1 = degp[0][:, :16], degp[1][:, :16]

    y1 = _t1(x, W1, d0, d1)
    p = edge_kernel(srcp, dstp, y1, z128)
    y2 = _t2(p[0], p[1], d0, d1, b1r, W2)
    p = edge_kernel(srcp, dstp, y2, z128)
    y3 = _t2(p[0], p[1], d0, d1, b2r, W3)
    p = edge_kernel(srcp, dstp, y3, z128)
    sums, cnts = _t3(p[0], p[1], d0, d1, b3r, batch3)
    return _t4(sums, cnts, Wl, blr)


def kernel(x, edge_index, batch, W1, b1, W2, b2, W3, b3, Wl, bl):
    return _run(x, edge_index, batch, W1, b1, W2, b2, W3, b3, Wl, bl)
